# Initial kernel scaffold; baseline (speedup 1.0000x reference)
import jax
import jax.numpy as jnp
from jax.experimental import pallas as pl


def kernel(heads, rels, tails, dateid, hiss, ent_hiss, ent_w, rel_w, tim_w):
    raise NotImplementedError("write your pallas kernel here")


# speedup vs baseline: 7.4782x; 7.4782x over previous
import functools

import jax
import jax.numpy as jnp
from jax import lax
from jax.experimental import pallas as pl
from jax.experimental.pallas import tpu as pltpu
from jax.experimental.pallas import tpu_sc as plsc

NUM_ENT = 100000
NUM_REL = 1000
NUM_TIME = 1000
EMB = 128
T_EMB = 64
ALP = 0.5
B = 16384
H = 50

NC = 2
NS = 16
NW = NC * NS
BPW = B // NW
CB = 128
NCHUNK = BPW // CB
NBLK = NW * NCHUNK
NV = EMB // 16
NTV = T_EMB // 16


_GATHER_DNUMS = lax.GatherDimensionNumbers(
    offset_dims=(), collapsed_slice_dims=(0,), start_index_map=(0,))


def _lane_gather(x, idx):
    return lax.gather(
        x, idx[:, None], _GATHER_DNUMS, slice_sizes=(1,),
        mode=lax.GatherScatterMode.PROMISE_IN_BOUNDS)


def _sc_body(ehiss_hbm, rhiss_hbm, heads_hbm, rels_hbm, tails_hbm,
             dateid_hbm, ent_w, rel_w, tim_w, out_hbm,
             ehiss_v, rhiss_v, heads_v, rels_v, tails_v, dateid_v,
             h_rows, t_rows, r_rows, t1_rows, g_ent, g_rel,
             ssq_v, scores_v, sem):
    wid = lax.axis_index("s") * NC + lax.axis_index("c")

    def chunk_body(c, _):
        blk = wid * NCHUNK + c
        pltpu.sync_copy(ehiss_hbm.at[blk], ehiss_v)
        pltpu.sync_copy(rhiss_hbm.at[blk], rhiss_v)
        pltpu.sync_copy(heads_hbm.at[blk], heads_v)
        pltpu.sync_copy(rels_hbm.at[blk], rels_v)
        pltpu.sync_copy(tails_hbm.at[blk], tails_v)
        pltpu.sync_copy(dateid_hbm.at[blk], dateid_v)
        pltpu.async_copy(ent_w.at[heads_v], h_rows, sem).wait()
        pltpu.async_copy(ent_w.at[tails_v], t_rows, sem).wait()
        pltpu.async_copy(rel_w.at[rels_v], r_rows, sem).wait()
        pltpu.async_copy(tim_w.at[dateid_v], t1_rows, sem).wait()

        def outer_body(j0, _):
            def elem_body(j1, ssq_vec):
                j = j0 * 16 + j1
                cp_e = pltpu.async_copy(ent_w.at[ehiss_v.at[j]], g_ent, sem)
                cp_r = pltpu.async_copy(rel_w.at[rhiss_v.at[j]], g_rel, sem)
                cp_e.wait()
                cp_r.wait()

                def red_body(i, accs):
                    return tuple(
                        accs[v] + g_ent[i, pl.ds(v * 16, 16)] for v in range(NV)
                    ) + tuple(
                        accs[NV + v] + g_rel[i, pl.ds(v * 16, 16)]
                        for v in range(NV)
                    )

                zero = jnp.zeros((16,), jnp.float32)
                accs = lax.fori_loop(0, H, red_body, (zero,) * (2 * NV))

                sE = ALP / H
                acc16 = jnp.zeros((16,), jnp.float32)
                for v in range(NV):
                    pv = sE * accs[v]
                    qv = sE * accs[NV + v]
                    hv = (1.0 - ALP) * h_rows[j, pl.ds(v * 16, 16)] + pv
                    tv = (1.0 - ALP) * t_rows[j, pl.ds(v * 16, 16)] + pv
                    rv = (1.0 - ALP) * r_rows[j, pl.ds(v * 16, 16)] + qv
                    prod = hv * rv * tv
                    if v < NTV:
                        prod = prod * t1_rows[j, pl.ds(v * 16, 16)]
                    acc16 = acc16 + prod * prod

                lane = lax.iota(jnp.int32, 16)
                for d in (1, 2, 4, 8):
                    acc16 = acc16 + _lane_gather(acc16, lane ^ d)
                return jnp.where(lane == j1, acc16, ssq_vec)

            ssq_vec = lax.fori_loop(
                0, 16, elem_body, jnp.zeros((16,), jnp.float32))
            ssq_v[pl.ds(j0 * 16, 16)] = ssq_vec
            return 0

        lax.fori_loop(0, CB // 16, outer_body, 0)

        for v in range(CB // 16):
            x = ssq_v[pl.ds(v * 16, 16)]
            bits = lax.bitcast_convert_type(x, jnp.int32)
            y = lax.bitcast_convert_type(
                lax.shift_right_logical(bits, 1) + 0x1FBD1DF6, jnp.float32)
            for _ in range(3):
                y = 0.5 * (y + x / y)
            scores_v[pl.ds(v * 16, 16)] = -y

        pltpu.sync_copy(scores_v, out_hbm.at[pl.ds(blk * CB, CB)])
        return 0

    lax.fori_loop(0, NCHUNK, chunk_body, 0)


@jax.jit
def kernel(heads, rels, tails, dateid, hiss, ent_hiss, ent_w, rel_w, tim_w):
    mesh = plsc.VectorSubcoreMesh(
        core_axis_name="c", subcore_axis_name="s",
        num_cores=NC, num_subcores=NS)
    run = pl.kernel(
        _sc_body,
        out_type=jax.ShapeDtypeStruct((B,), jnp.float32),
        mesh=mesh,
        scratch_types=[
            pltpu.VMEM((CB, H), jnp.int32),
            pltpu.VMEM((CB, H), jnp.int32),
            pltpu.VMEM((CB,), jnp.int32),
            pltpu.VMEM((CB,), jnp.int32),
            pltpu.VMEM((CB,), jnp.int32),
            pltpu.VMEM((CB,), jnp.int32),
            pltpu.VMEM((CB, EMB), jnp.float32),
            pltpu.VMEM((CB, EMB), jnp.float32),
            pltpu.VMEM((CB, EMB), jnp.float32),
            pltpu.VMEM((CB, EMB), jnp.float32),
            pltpu.VMEM((H, EMB), jnp.float32),
            pltpu.VMEM((H, EMB), jnp.float32),
            pltpu.VMEM((CB,), jnp.float32),
            pltpu.VMEM((CB,), jnp.float32),
            pltpu.SemaphoreType.DMA,
        ],
    )
    tim_w_pad = jnp.pad(tim_w, ((0, 0), (0, EMB - T_EMB)))
    ehiss_r = ent_hiss.reshape(NBLK, CB, H)
    rhiss_r = hiss.reshape(NBLK, CB, H)
    heads_r = heads.reshape(NBLK, CB)
    rels_r = rels.reshape(NBLK, CB)
    tails_r = tails.reshape(NBLK, CB)
    dateid_r = dateid.reshape(NBLK, CB)
    return run(ehiss_r, rhiss_r, heads_r, rels_r, tails_r, dateid_r,
               ent_w, rel_w, tim_w_pad)


# speedup vs baseline: 13.1901x; 1.7638x over previous
import functools

import jax
import jax.numpy as jnp
from jax import lax
from jax.experimental import pallas as pl
from jax.experimental.pallas import tpu as pltpu
from jax.experimental.pallas import tpu_sc as plsc

NUM_ENT = 100000
NUM_REL = 1000
NUM_TIME = 1000
EMB = 128
T_EMB = 64
ALP = 0.5
B = 16384
H = 50

NC = 2
NS = 16
NW = NC * NS
BPW = B // NW
CB = 128
NCHUNK = BPW // CB
NBLK = NW * NCHUNK
NV = EMB // 16
NTV = T_EMB // 16


_GATHER_DNUMS = lax.GatherDimensionNumbers(
    offset_dims=(), collapsed_slice_dims=(0,), start_index_map=(0,))


def _lane_gather(x, idx):
    return lax.gather(
        x, idx[:, None], _GATHER_DNUMS, slice_sizes=(1,),
        mode=lax.GatherScatterMode.PROMISE_IN_BOUNDS)


def _sc_body(ehiss_hbm, rhiss_hbm, heads_hbm, rels_hbm, tails_hbm,
             dateid_hbm, ent_w, rel_w, tim_w, out_hbm,
             ehiss_v, rhiss_v, heads_v, rels_v, tails_v, dateid_v,
             h_rows, t_rows, r_rows, t1_rows, g_ent, g_rel,
             ssq_v, scores_v, sems):
    wid = lax.axis_index("s") * NC + lax.axis_index("c")

    def issue_elem(j, pb):
        pltpu.async_copy(ent_w.at[ehiss_v.at[j]], g_ent.at[pb], sems.at[pb])
        pltpu.async_copy(rel_w.at[rhiss_v.at[j]], g_rel.at[pb], sems.at[pb])

    def wait_elem(j, pb):
        pltpu.make_async_copy(
            ent_w.at[ehiss_v.at[j]], g_ent.at[pb], sems.at[pb]).wait()
        pltpu.make_async_copy(
            rel_w.at[rhiss_v.at[j]], g_rel.at[pb], sems.at[pb]).wait()

    def chunk_body(c, _):
        blk = wid * NCHUNK + c
        pltpu.sync_copy(ehiss_hbm.at[blk], ehiss_v)
        pltpu.sync_copy(rhiss_hbm.at[blk], rhiss_v)
        pltpu.sync_copy(heads_hbm.at[blk], heads_v)
        pltpu.sync_copy(rels_hbm.at[blk], rels_v)
        pltpu.sync_copy(tails_hbm.at[blk], tails_v)
        pltpu.sync_copy(dateid_hbm.at[blk], dateid_v)
        cp1 = pltpu.async_copy(ent_w.at[heads_v], h_rows, sems.at[0])
        cp2 = pltpu.async_copy(ent_w.at[tails_v], t_rows, sems.at[0])
        cp3 = pltpu.async_copy(rel_w.at[rels_v], r_rows, sems.at[0])
        cp4 = pltpu.async_copy(tim_w.at[dateid_v], t1_rows, sems.at[0])
        cp1.wait()
        cp2.wait()
        cp3.wait()
        cp4.wait()

        issue_elem(0, 0)

        def outer_body(j0, _):
            def elem_body(j1, ssq_vec):
                j = j0 * 16 + j1
                p = j & 1
                @pl.when(j < CB - 1)
                def _():
                    issue_elem(j + 1, 1 - p)

                wait_elem(j, p)

                def red_body(i, accs):
                    i2 = i * 2
                    return tuple(
                        accs[v] + (g_ent[p, i2, pl.ds(v * 16, 16)]
                                   + g_ent[p, i2 + 1, pl.ds(v * 16, 16)])
                        for v in range(NV)
                    ) + tuple(
                        accs[NV + v] + (g_rel[p, i2, pl.ds(v * 16, 16)]
                                        + g_rel[p, i2 + 1, pl.ds(v * 16, 16)])
                        for v in range(NV)
                    )

                zero = jnp.zeros((16,), jnp.float32)
                accs = lax.fori_loop(0, H // 2, red_body, (zero,) * (2 * NV))

                sE = ALP / H
                acc16 = jnp.zeros((16,), jnp.float32)
                for v in range(NV):
                    pv = sE * accs[v]
                    qv = sE * accs[NV + v]
                    hv = (1.0 - ALP) * h_rows[j, pl.ds(v * 16, 16)] + pv
                    tv = (1.0 - ALP) * t_rows[j, pl.ds(v * 16, 16)] + pv
                    rv = (1.0 - ALP) * r_rows[j, pl.ds(v * 16, 16)] + qv
                    prod = hv * rv * tv
                    if v < NTV:
                        prod = prod * t1_rows[j, pl.ds(v * 16, 16)]
                    acc16 = acc16 + prod * prod

                lane = lax.iota(jnp.int32, 16)
                for d in (1, 2, 4, 8):
                    acc16 = acc16 + _lane_gather(acc16, lane ^ d)
                return jnp.where(lane == j1, acc16, ssq_vec)

            ssq_vec = lax.fori_loop(
                0, 16, elem_body, jnp.zeros((16,), jnp.float32))
            ssq_v[pl.ds(j0 * 16, 16)] = ssq_vec
            return 0

        lax.fori_loop(0, CB // 16, outer_body, 0)

        for v in range(CB // 16):
            x = ssq_v[pl.ds(v * 16, 16)]
            bits = lax.bitcast_convert_type(x, jnp.int32)
            y = lax.bitcast_convert_type(
                lax.shift_right_logical(bits, 1) + 0x1FBD1DF6, jnp.float32)
            for _ in range(3):
                y = 0.5 * (y + x / y)
            scores_v[pl.ds(v * 16, 16)] = -y

        pltpu.sync_copy(scores_v, out_hbm.at[pl.ds(blk * CB, CB)])
        return 0

    lax.fori_loop(0, NCHUNK, chunk_body, 0)


@jax.jit
def kernel(heads, rels, tails, dateid, hiss, ent_hiss, ent_w, rel_w, tim_w):
    mesh = plsc.VectorSubcoreMesh(
        core_axis_name="c", subcore_axis_name="s",
        num_cores=NC, num_subcores=NS)
    run = pl.kernel(
        _sc_body,
        out_type=jax.ShapeDtypeStruct((B,), jnp.float32),
        mesh=mesh,
        scratch_types=[
            pltpu.VMEM((CB, H), jnp.int32),
            pltpu.VMEM((CB, H), jnp.int32),
            pltpu.VMEM((CB,), jnp.int32),
            pltpu.VMEM((CB,), jnp.int32),
            pltpu.VMEM((CB,), jnp.int32),
            pltpu.VMEM((CB,), jnp.int32),
            pltpu.VMEM((CB, EMB), jnp.float32),
            pltpu.VMEM((CB, EMB), jnp.float32),
            pltpu.VMEM((CB, EMB), jnp.float32),
            pltpu.VMEM((CB, EMB), jnp.float32),
            pltpu.VMEM((2, H, EMB), jnp.float32),
            pltpu.VMEM((2, H, EMB), jnp.float32),
            pltpu.VMEM((CB,), jnp.float32),
            pltpu.VMEM((CB,), jnp.float32),
            pltpu.SemaphoreType.DMA((2,)),
        ],
    )
    tim_w_pad = jnp.pad(tim_w, ((0, 0), (0, EMB - T_EMB)))
    ehiss_r = ent_hiss.reshape(NBLK, CB, H)
    rhiss_r = hiss.reshape(NBLK, CB, H)
    heads_r = heads.reshape(NBLK, CB)
    rels_r = rels.reshape(NBLK, CB)
    tails_r = tails.reshape(NBLK, CB)
    dateid_r = dateid.reshape(NBLK, CB)
    return run(ehiss_r, rhiss_r, heads_r, rels_r, tails_r, dateid_r,
               ent_w, rel_w, tim_w_pad)
